# baseline (speedup 1.0000x reference)
import jax
import jax.numpy as jnp
from jax.experimental import pallas as pl

ROWS, COLS = 8192, 4096
SHIFT = 1024
KEEP = ROWS - SHIFT
BLK = 512
GRID = ROWS // BLK
KEEP_BLKS = KEEP // BLK
SHIFT_BLKS = SHIFT // BLK
YBLK = SHIFT // GRID


def _body(shift_src, y_src, x_src, out_ref, y_ref):
    i = pl.program_id(0)

    @pl.when(i < KEEP_BLKS)
    def _():
        out_ref[...] = shift_src[...]

    @pl.when(i >= KEEP_BLKS)
    def _():
        out_ref[...] = x_src[...]

    y_ref[...] = y_src[...]


def kernel(buffer, x):
    out_buf, y = pl.pallas_call(
        _body,
        grid=(GRID,),
        in_specs=[
            pl.BlockSpec((BLK, COLS),
                         lambda i: (jnp.minimum(i + SHIFT_BLKS, GRID - 1), 0)),
            pl.BlockSpec((YBLK, COLS), lambda i: (i, 0)),
            pl.BlockSpec((BLK, COLS),
                         lambda i: (jnp.clip(i - KEEP_BLKS, 0, SHIFT_BLKS - 1), 0)),
        ],
        out_specs=[
            pl.BlockSpec((BLK, COLS), lambda i: (i, 0)),
            pl.BlockSpec((YBLK, COLS), lambda i: (i, 0)),
        ],
        out_shape=[
            jax.ShapeDtypeStruct((ROWS, COLS), jnp.float32),
            jax.ShapeDtypeStruct((SHIFT, COLS), jnp.float32),
        ],
    )(buffer, buffer, x)
    return (out_buf, y)
